# baseline (speedup 1.0000x reference)
import functools

import jax
import jax.numpy as jnp
import numpy as np
from jax import lax
from jax.experimental import pallas as pl
from jax.experimental.pallas import tpu as pltpu
from jax.experimental.pallas import tpu_sc as plsc

NUM_POS = 4096
NUM_NEG = 999
TOP_N = 10
SIZE = NUM_POS * (NUM_NEG + 1)
ROW_PAD = 1024
PAD_SIZE = NUM_POS * ROW_PAD

NC, NS, L = 2, 16, 16
NW = NC * NS
CHUNK = SIZE // NW
LSUB = CHUNK // L
WIN_L = 1000
WIN = WIN_L * L
NWIN = LSUB // WIN_L
CPW = WIN // 128

PASSES = ((0, 11), (11, 11), (22, 10))

@functools.cache
def _mesh():
    return plsc.VectorSubcoreMesh(
        core_axis_name="c", subcore_axis_name="s", num_cores=NC, num_subcores=NS
    )


def _wid():
    return lax.axis_index("s") * NC + lax.axis_index("c")


def _monotone_key(x_i32):
    m = lax.shift_right_arithmetic(x_i32, 31)
    return lax.bitwise_xor(x_i32, lax.bitwise_or(m, jnp.int32(-2147483648)))


def _srl(x, n):
    return lax.shift_right_logical(x, jnp.int32(n))


def _make_phase_a(shift, bits, src_is_f32):
    nb = 1 << bits
    mask = jnp.int32(nb - 1)

    def body(src, cnt3_out, cnt_out, win, hist3, cntrow, sem):
        wid = _wid()
        iota = lax.iota(jnp.int32, L)
        lane_off = iota * WIN_L
        ones = jnp.ones((L,), jnp.int32)
        zeros16 = jnp.zeros((L,), jnp.int32)

        def zero_body(i, _):
            hist3[pl.ds(i * L, L)] = zeros16
            return 0

        lax.fori_loop(0, nb * L // L, zero_body, 0)

        def win_desc(w, l):
            return pltpu.make_async_copy(
                src.at[pl.ds(wid * CHUNK + l * LSUB + w * WIN_L, WIN_L)],
                win.at[pl.ds(l * WIN_L, WIN_L)],
                sem,
            )

        def window(w, _):
            lax.fori_loop(0, L, lambda l, c: (win_desc(w, l).start(), c)[1], 0)
            lax.fori_loop(0, L, lambda l, c: (win_desc(w, l).wait(), c)[1], 0)

            def col(i, _):
                idxv = lane_off + i
                x = plsc.load_gather(win, [idxv])
                if src_is_f32:
                    k = _monotone_key(plsc.bitcast(x, jnp.int32))
                else:
                    k = x
                dg = lax.bitwise_and(_srl(k, shift), mask)
                slot = dg * L + iota
                plsc.addupdate_scatter(hist3, [slot], ones)
                return 0

            lax.fori_loop(0, WIN_L, col, 0)
            return 0

        lax.fori_loop(0, NWIN, window, 0)

        def reduce_body(dc, _):
            dig16 = (dc * L + iota) * L
            accv = zeros16
            for j in range(L):
                accv = accv + plsc.load_gather(hist3, [dig16 + j])
            cntrow[pl.ds(dc * L, L)] = accv
            return 0

        lax.fori_loop(0, nb // L, reduce_body, 0)

        pltpu.sync_copy(hist3, cnt3_out.at[wid])
        pltpu.sync_copy(cntrow, cnt_out.at[wid])

    src_dtype = jnp.float32 if src_is_f32 else jnp.int32
    return pl.kernel(
        body,
        out_type=[
            jax.ShapeDtypeStruct((NW, nb * L), jnp.int32),
            jax.ShapeDtypeStruct((NW, nb), jnp.int32),
        ],
        mesh=_mesh(),
        compiler_params=pltpu.CompilerParams(use_tc_tiling_on_sc=False, needs_layout_passes=False),
        scratch_types=[
            pltpu.VMEM((WIN,), src_dtype),
            pltpu.VMEM((nb * L,), jnp.int32),
            pltpu.VMEM((nb,), jnp.int32),
            pltpu.SemaphoreType.DMA,
        ],
        name=f"radix_hist_s{shift}",
    )


def _make_phase_c(shift, bits, src_is_f32, final):
    nb = 1 << bits
    mask = jnp.int32(nb - 1)
    src_dtype = jnp.float32 if src_is_f32 else jnp.int32

    def body(src_k, src_v, cnt3, cnt, *rest):
        if final:
            (dst_v, win_k, win_v, win_ko, didx, base3, rowbuf, acc, tot,
             sem, sem2) = rest
            dst_k = None
        else:
            (dst_k, dst_v, win_k, win_v, win_ko, didx, base3, rowbuf, acc, tot,
             sem, sem2) = rest
        wid = _wid()
        iota = lax.iota(jnp.int32, L)
        lane_off = iota * WIN_L
        zeros16 = jnp.zeros((L,), jnp.int32)

        def zb(i, _):
            acc[pl.ds(i * L, L)] = zeros16
            tot[pl.ds(i * L, L)] = zeros16
            return 0

        lax.fori_loop(0, nb // L, zb, 0)

        def worker_row(w2, _):
            pltpu.sync_copy(cnt.at[w2], rowbuf)
            sel = jnp.where(w2 < wid, jnp.int32(1), jnp.int32(0))

            def addrow(db, _):
                v = rowbuf[pl.ds(db * L, L)]
                tot[pl.ds(db * L, L)] = tot[pl.ds(db * L, L)] + v
                acc[pl.ds(db * L, L)] = acc[pl.ds(db * L, L)] + v * sel
                return 0

            lax.fori_loop(0, nb // L, addrow, 0)
            return 0

        lax.fori_loop(0, NW, worker_row, 0)

        def scan_body(db, carry):
            v = tot[pl.ds(db * L, L)]
            inc = plsc.cumsum(v)
            tot[pl.ds(db * L, L)] = inc - v + carry
            return carry + jnp.sum(v, axis=0)

        lax.fori_loop(0, nb // L, scan_body, jnp.int32(0))

        pltpu.sync_copy(cnt3.at[wid], base3)

        def baserow(d, _):
            row = base3[pl.ds(d * L, L)]
            ex = plsc.cumsum(row) - row
            bcast_d = iota * 0 + d
            splat = plsc.load_gather(tot, [bcast_d]) + plsc.load_gather(acc, [bcast_d])
            base3[pl.ds(d * L, L)] = ex + splat
            return 0

        lax.fori_loop(0, nb, baserow, 0)

        def load_desc(w, l, which):
            src = src_k if which == 0 else src_v
            dst = win_k if which == 0 else win_v
            return pltpu.make_async_copy(
                src.at[pl.ds(wid * CHUNK + l * LSUB + w * WIN_L, WIN_L)],
                dst.at[pl.ds(l * WIN_L, WIN_L)],
                sem,
            )

        def scat_desc(w, c, which):
            if which == 0:
                return pltpu.make_async_copy(
                    win_v.at[pl.ds(c * 128, 128)], dst_v.at[didx.at[c]], sem2
                )
            return pltpu.make_async_copy(
                win_ko.at[pl.ds(c * 128, 128)], dst_k.at[didx.at[c]], sem2
            )

        def window(w, _):
            def lstart(l, c):
                load_desc(w, l, 0).start()
                load_desc(w, l, 1).start()
                return c

            def lwait(l, c):
                load_desc(w, l, 0).wait()
                load_desc(w, l, 1).wait()
                return c

            lax.fori_loop(0, L, lstart, 0)
            lax.fori_loop(0, L, lwait, 0)

            def col(i, _):
                idxv = lane_off + i
                x = plsc.load_gather(win_k, [idxv])
                if src_is_f32:
                    k = _monotone_key(plsc.bitcast(x, jnp.int32))
                else:
                    k = x
                if not final:
                    plsc.store_scatter(win_ko, [idxv], k)
                dg = lax.bitwise_and(_srl(k, shift), mask)
                slot = dg * L + iota
                g = plsc.load_gather(base3, [slot])
                plsc.store_scatter(base3, [slot], g + 1)
                if final:
                    i_neg = g - NUM_POS
                    r = i_neg // NUM_NEG
                    dest = jnp.where(
                        g < NUM_POS,
                        g * ROW_PAD + NUM_NEG,
                        r * ROW_PAD + (i_neg - r * NUM_NEG),
                    )
                else:
                    dest = g
                plsc.store_scatter(didx, [_srl(idxv, 7), lax.bitwise_and(idxv, jnp.int32(127))], dest)
                return 0

            lax.fori_loop(0, WIN_L, col, 0)

            def sstart(c, a):
                scat_desc(w, c, 0).start()
                if not final:
                    scat_desc(w, c, 1).start()
                return a

            def swait(c, a):
                scat_desc(w, c, 0).wait()
                if not final:
                    scat_desc(w, c, 1).wait()
                return a

            lax.fori_loop(0, CPW, sstart, 0)
            lax.fori_loop(0, CPW, swait, 0)
            return 0

        lax.fori_loop(0, NWIN, window, 0)

    out_type = [jax.ShapeDtypeStruct((PAD_SIZE,), jnp.float32)] if final else [
        jax.ShapeDtypeStruct((SIZE,), jnp.int32),
        jax.ShapeDtypeStruct((SIZE,), jnp.float32),
    ]
    return pl.kernel(
        body,
        out_type=out_type,
        mesh=_mesh(),
        compiler_params=pltpu.CompilerParams(use_tc_tiling_on_sc=False, needs_layout_passes=False),
        scratch_types=[
            pltpu.VMEM((WIN,), src_dtype),
            pltpu.VMEM((WIN,), jnp.float32),
            pltpu.VMEM((WIN,), jnp.int32),
            pltpu.VMEM((CPW, 128), jnp.int32),
            pltpu.VMEM((nb * L,), jnp.int32),
            pltpu.VMEM((nb,), jnp.int32),
            pltpu.VMEM((nb,), jnp.int32),
            pltpu.VMEM((nb,), jnp.int32),
            pltpu.SemaphoreType.DMA,
            pltpu.SemaphoreType.DMA,
        ],
        name=f"radix_scatter_s{shift}",
    )


@functools.cache
def _sort_kernels():
    a1 = _make_phase_a(*PASSES[0][:2], True)
    c1 = _make_phase_c(*PASSES[0][:2], True, False)
    a2 = _make_phase_a(*PASSES[1][:2], False)
    c2 = _make_phase_c(*PASSES[1][:2], False, False)
    a3 = _make_phase_a(*PASSES[2][:2], False)
    c3 = _make_phase_c(*PASSES[2][:2], False, True)
    return a1, c1, a2, c2, a3, c3


def _sc_sort(index, predict_val):
    a1, c1, a2, c2, a3, c3 = _sort_kernels()
    cnt3, cnt = a1(index)
    k1, v1 = c1(index, predict_val, cnt3, cnt)
    cnt3, cnt = a2(k1)
    k2, v2 = c2(k1, v1, cnt3, cnt)
    cnt3, cnt = a3(k2)
    (pv_pad,) = (c3(k2, v2, cnt3, cnt),)
    if isinstance(pv_pad, (tuple, list)):
        pv_pad = pv_pad[0]
    return pv_pad.reshape(NUM_POS, ROW_PAD)


def _threefry2x32(ks0, ks1, x0, x1):
    u32 = np.uint32

    def rotl(x, d):
        return ((x << u32(d)) | (x >> u32(32 - d))).astype(u32)

    ks2 = (ks0 ^ ks1 ^ u32(0x1BD11BDA)).astype(u32)
    ks = [ks0, ks1, ks2]
    rotations = [(13, 15, 26, 6), (17, 29, 16, 24)]
    x0 = (x0 + ks[0]).astype(u32)
    x1 = (x1 + ks[1]).astype(u32)
    for i in range(5):
        for r in rotations[i % 2]:
            x0 = (x0 + x1).astype(u32)
            x1 = rotl(x1, r)
            x1 = (x1 ^ x0).astype(u32)
        x0 = (x0 + ks[(i + 1) % 3]).astype(u32)
        x1 = (x1 + ks[(i + 2) % 3] + u32(i + 1)).astype(u32)
    return x0, x1


def _reference_perms() -> np.ndarray:
    u32 = np.uint32
    n = NUM_NEG + 1
    zeros = np.zeros(NUM_POS, u32)
    k0, k1 = _threefry2x32(zeros, np.full(NUM_POS, 42, u32),
                           zeros, np.arange(NUM_POS, dtype=u32))
    s0, s1 = _threefry2x32(k0, k1, zeros, np.ones(NUM_POS, u32))
    a, b = _threefry2x32(
        s0[:, None], s1[:, None],
        np.zeros((NUM_POS, n), u32), np.broadcast_to(np.arange(n, dtype=u32), (NUM_POS, n)),
    )
    bits = (a ^ b).astype(u32)
    return np.argsort(bits, axis=1, kind="stable").astype(np.int32)


@functools.cache
def _row_masks() -> np.ndarray:
    perms = _reference_perms()
    inv = np.argsort(perms, axis=1)
    v = perms[:, -1]
    p = inv[:, NUM_NEG]
    m = np.ones((NUM_POS, NUM_NEG), dtype=np.float32)
    m[inv[:, :NUM_NEG] < p[:, None]] = 2.0
    full = v == NUM_NEG
    m[full, :] = 2.0
    rows = np.where(~full)[0]
    m[rows, v[~full]] = 0.0
    padded = np.zeros((NUM_POS, ROW_PAD), dtype=np.float32)
    padded[:, :NUM_NEG] = m
    return padded


_MASKS = _row_masks()

ROWS_PER_BLK = 128
GRID = NUM_POS // ROWS_PER_BLK


def _metric_body(pv_ref, m_ref, hit_ref, ndcg_ref):
    pv = pv_ref[...]
    m = m_ref[...]
    pos = pv[:, NUM_NEG:NUM_NEG + 1]
    gt = jnp.logical_and(pv > pos, m > 0.5)
    ge = jnp.logical_and(pv == pos, m > 1.5)
    cnt = jnp.sum(gt.astype(jnp.float32) + ge.astype(jnp.float32), axis=1)
    rank = cnt + 1.0
    hit = (rank <= TOP_N).astype(jnp.float32)
    ndcg = hit / jnp.log2(rank + 1.0)
    hit_ref[...] = hit[None, None, :]
    ndcg_ref[...] = ndcg[None, None, :]


def _metrics(pv_pad):
    m = jnp.asarray(_MASKS)
    hit, ndcg = pl.pallas_call(
        _metric_body,
        grid=(GRID,),
        in_specs=[
            pl.BlockSpec((ROWS_PER_BLK, ROW_PAD), lambda i: (i, 0)),
            pl.BlockSpec((ROWS_PER_BLK, ROW_PAD), lambda i: (i, 0)),
        ],
        out_specs=[
            pl.BlockSpec((1, 1, ROWS_PER_BLK), lambda i: (i, 0, 0)),
            pl.BlockSpec((1, 1, ROWS_PER_BLK), lambda i: (i, 0, 0)),
        ],
        out_shape=[
            jax.ShapeDtypeStruct((GRID, 1, ROWS_PER_BLK), jnp.float32),
            jax.ShapeDtypeStruct((GRID, 1, ROWS_PER_BLK), jnp.float32),
        ],
    )(pv_pad, m)
    return hit.reshape(NUM_POS), ndcg.reshape(NUM_POS)


def kernel(n, num, predict_val, num_pos, index):
    pv_pad = _sc_sort(index, predict_val)
    hits, ndcgs = _metrics(pv_pad)
    Hits = jnp.sum(hits) / num_pos
    ndcg = jnp.sum(ndcgs) / num_pos
    return Hits, ndcg, hits, ndcgs
